# baseline (speedup 1.0000x reference)
import functools
import math

import jax
import jax.numpy as jnp
from jax import lax
from jax.experimental import pallas as pl
from jax.experimental.pallas import tpu as pltpu
from jax.experimental.pallas import tpu_sc as plsc

D_MODEL = 64
BATCH = 16384
SEQ = 50
NB = BATCH * SEQ
NC, NS, L = 2, 16, 16
NW = NC * NS
KI = 128
GPC = 4
K = KI * GPC
BPW = NB // NW
NCHUNK = BPW // K
SCALE = math.sqrt(D_MODEL)

_mesh = plsc.VectorSubcoreMesh(core_axis_name="c", subcore_axis_name="s")


@functools.partial(
    pl.kernel,
    out_type=jax.ShapeDtypeStruct((NB, D_MODEL), jnp.float32),
    mesh=_mesh,
    scratch_types=[
        pltpu.VMEM((K,), jnp.int32),
        pltpu.VMEM((K, D_MODEL), jnp.float32),
        pltpu.SemaphoreType.DMA,
    ],
    compiler_params=pltpu.CompilerParams(use_tc_tiling_on_sc=False),
)
def _embed(x_hbm, tab_hbm, out_hbm, idx_v, rows_v, sem):
    wid = lax.axis_index("s") * NC + lax.axis_index("c")
    base = wid * BPW

    def chunk(g, carry):
        cb = base + g * K
        pltpu.sync_copy(x_hbm.at[pl.ds(cb, K)], idx_v)
        for j in range(GPC):
            pltpu.async_copy(
                tab_hbm.at[idx_v.at[pl.ds(j * KI, KI)]],
                rows_v.at[pl.ds(j * KI, KI)],
                sem,
            )
        for j in range(GPC):
            pltpu.make_async_copy(
                tab_hbm.at[idx_v.at[pl.ds(j * KI, KI)]],
                rows_v.at[pl.ds(j * KI, KI)],
                sem,
            ).wait()

        def srow(r, c):
            for cc in range(D_MODEL // L):
                rows_v[r, pl.ds(cc * L, L)] = rows_v[r, pl.ds(cc * L, L)] * SCALE
            return c

        lax.fori_loop(0, K, srow, 0)
        pltpu.sync_copy(rows_v, out_hbm.at[pl.ds(cb, K)])
        return carry

    lax.fori_loop(0, NCHUNK, chunk, 0)


def kernel(x, table):
    xi = x.reshape(NB).astype(jnp.int32)
    out = _embed(xi, table)
    return out.reshape(BATCH, SEQ, D_MODEL)


# speedup vs baseline: 1.1233x; 1.1233x over previous
import functools
import math

import jax
import jax.numpy as jnp
from jax import lax
from jax.experimental import pallas as pl
from jax.experimental.pallas import tpu as pltpu
from jax.experimental.pallas import tpu_sc as plsc

D_MODEL = 64
BATCH = 16384
SEQ = 50
NB = BATCH * SEQ
NC, NS, L = 2, 16, 16
NW = NC * NS
KI = 128
GPC = 2
K = KI * GPC
NBUF = 4
BPW = NB // NW
NCHUNK = BPW // K
RU = 4
SCALE = math.sqrt(D_MODEL)

_mesh = plsc.VectorSubcoreMesh(core_axis_name="c", subcore_axis_name="s")


@functools.partial(
    pl.kernel,
    out_type=jax.ShapeDtypeStruct((NB, D_MODEL), jnp.float32),
    mesh=_mesh,
    scratch_types=[
        [pltpu.VMEM((K,), jnp.int32) for _ in range(NBUF)],
        [pltpu.VMEM((K, D_MODEL), jnp.float32) for _ in range(NBUF)],
        pltpu.SemaphoreType.DMA,
        pltpu.SemaphoreType.DMA,
    ],
    compiler_params=pltpu.CompilerParams(use_tc_tiling_on_sc=False),
)
def _embed(x_hbm, tab_hbm, out_hbm, idx_bufs, row_bufs, gsem, ssem):
    wid = lax.axis_index("s") * NC + lax.axis_index("c")
    base = wid * BPW

    def fire_gather(c, b):
        iv, rv = idx_bufs[b], row_bufs[b]
        pltpu.sync_copy(x_hbm.at[pl.ds(base + c * K, K)], iv)
        for j in range(GPC):
            pltpu.async_copy(
                tab_hbm.at[iv.at[pl.ds(j * KI, KI)]],
                rv.at[pl.ds(j * KI, KI)],
                gsem,
            )

    def wait_gather(b):
        iv, rv = idx_bufs[b], row_bufs[b]
        for j in range(GPC):
            pltpu.make_async_copy(
                tab_hbm.at[iv.at[pl.ds(j * KI, KI)]],
                rv.at[pl.ds(j * KI, KI)],
                gsem,
            ).wait()

    def drain_store(b):
        pltpu.make_async_copy(row_bufs[b], out_hbm.at[pl.ds(base, K)], ssem).wait()

    fire_gather(0, 0)
    fire_gather(1, 1)

    def step(p, carry):
        for b in range(NBUF):
            c = p * NBUF + b
            rv = row_bufs[b]
            wait_gather(b)

            def srows(r, cc):
                for u in range(RU):
                    for col in range(D_MODEL // L):
                        sl = pl.ds(col * L, L)
                        rv[r + u, sl] = rv[r + u, sl] * SCALE
                return cc

            lax.fori_loop(0, K // RU, lambda r, cc: srows(r * RU, cc), 0)

            pltpu.async_copy(rv, out_hbm.at[pl.ds(base + c * K, K)], ssem)

            nxt = c + 2
            bn = (b + 2) % NBUF

            @pl.when(nxt < NCHUNK)
            def _():
                @pl.when(c >= 2)
                def _():
                    drain_store(bn)

                fire_gather(nxt, bn)

        return carry

    lax.fori_loop(0, NCHUNK // NBUF, step, 0)
    for b in range(NBUF):
        drain_store(b)


def kernel(x, table):
    xi = x.reshape(NB).astype(jnp.int32)
    out = _embed(xi, table)
    return out.reshape(BATCH, SEQ, D_MODEL)
